# baseline (speedup 1.0000x reference)
import functools
from typing import Any

import jax
import jax.numpy as jnp
from jax import lax
from jax.experimental import pallas as pl
from jax.experimental.pallas import tpu as pltpu

B = 4
V = 8
C = 256
N = 196
T1 = V - 1
K = 8
NODES = V * N
E = 2 * T1 * K
CNT = B * V * N
NPAIR = B * T1
ROWLEN = N * N
NVREG = ROWLEN // 16
_BIGI = 2**30


def _tca_body(xm_ref, xc_ref, xp_ref, w_ref, g_ref, be_ref,
              y_ref, sco_ref, sho_ref, sm_ref, ys_ref, st_ref):
    i = pl.program_id(0)

    @pl.when(i == 0)
    def _():
        st_ref[...] = jnp.zeros_like(st_ref)

    @pl.when(i < B * V)
    def _():
        t = i % V
        xm = xm_ref[0]
        xc = xc_ref[0]
        xp = xp_ref[0]
        w = w_ref[...]
        dn = (((0,), (1,)), ((), ()))
        y = lax.dot_general(xc, w[1], dn, preferred_element_type=jnp.float32)
        ym = lax.dot_general(xm, w[0], dn, preferred_element_type=jnp.float32)
        yp = lax.dot_general(xp, w[2], dn, preferred_element_type=jnp.float32)
        mm = jnp.where(t > 0, 1.0, 0.0).astype(jnp.float32)
        mp = jnp.where(t < V - 1, 1.0, 0.0).astype(jnp.float32)
        y = y + mm * ym + mp * yp
        y_ref[0, 0] = y
        ys_ref[pl.ds(i, 1)] = y[None]
        st_ref[0:1, :] += jnp.sum(y, axis=0, keepdims=True)
        st_ref[1:2, :] += jnp.sum(y * y, axis=0, keepdims=True)

    @pl.when(i == B * V)
    def _():
        eps = jnp.float32(1e-5)
        mean = st_ref[0:1, :] / CNT
        var = st_ref[1:2, :] / CNT - mean * mean
        s = lax.rsqrt(var + eps) * g_ref[...]
        sh = be_ref[...] - mean * s
        st_ref[2:3, :] = s
        st_ref[3:4, :] = sh
        sco_ref[...] = s
        sho_ref[...] = sh

    @pl.when(i >= B * V)
    def _():
        p = i - B * V
        b = p // T1
        t = p % T1
        q = b * V + t
        s = st_ref[2:3, :]
        sh = st_ref[3:4, :]
        xa = ys_ref[pl.ds(q, 1)][0] * s + sh
        xb = ys_ref[pl.ds(q + 1, 1)][0] * s + sh
        a2 = jnp.sum(xa * xa, axis=1, keepdims=True)
        b2 = jnp.sum(xb * xb, axis=1, keepdims=True)
        g = lax.dot_general(xa, xb, (((1,), (1,)), ((), ())),
                            preferred_element_type=jnp.float32)
        d2 = jnp.clip(a2 + b2.T - 2.0 * g, 0.0, None)
        sim = -jnp.sqrt(d2)
        nrm = jnp.sqrt(jnp.sum(sim * sim))
        sm_ref[0] = sim / jnp.maximum(nrm, 1e-12)


def _tca(x3, w3, bn1_g2, bn1_b2):
    nsteps = B * V + NPAIR
    zero3 = lambda i: (0, 0, 0)
    return pl.pallas_call(
        _tca_body,
        grid=(nsteps,),
        in_specs=[
            pl.BlockSpec((1, C, N), lambda i: (
                jnp.where(i < B * V,
                          (i // V) * V + jnp.maximum(i % V - 1, 0), 0), 0, 0)),
            pl.BlockSpec((1, C, N), lambda i: (jnp.where(i < B * V, i, 0), 0, 0)),
            pl.BlockSpec((1, C, N), lambda i: (
                jnp.where(i < B * V,
                          (i // V) * V + jnp.minimum(i % V + 1, V - 1), 0), 0, 0)),
            pl.BlockSpec((3, C, C), zero3),
            pl.BlockSpec((1, C), lambda i: (0, 0)),
            pl.BlockSpec((1, C), lambda i: (0, 0)),
        ],
        out_specs=[
            pl.BlockSpec((1, 1, N, C), lambda i: (
                jnp.where(i < B * V, i // V, B - 1),
                jnp.where(i < B * V, i % V, V - 1), 0, 0)),
            pl.BlockSpec((1, C), lambda i: (0, 0)),
            pl.BlockSpec((1, C), lambda i: (0, 0)),
            pl.BlockSpec((1, N, N), lambda i: (
                jnp.where(i < B * V, 0, i - B * V), 0, 0)),
        ],
        out_shape=[
            jax.ShapeDtypeStruct((B, V, N, C), jnp.float32),
            jax.ShapeDtypeStruct((1, C), jnp.float32),
            jax.ShapeDtypeStruct((1, C), jnp.float32),
            jax.ShapeDtypeStruct((NPAIR, N, N), jnp.float32),
        ],
        scratch_shapes=[
            pltpu.VMEM((B * V, N, C), jnp.float32),
            pltpu.VMEM((8, C), jnp.float32),
        ],
    )(x3, x3, x3, w3, bn1_g2, bn1_b2)


def _sc_insert(kregs, iregs, kv, iv):
    for j in range(K):
        c = kv > kregs[j]
        nk = jnp.where(c, kv, kregs[j])
        kv = jnp.where(c, kregs[j], kv)
        ni = jnp.where(c, iv, iregs[j])
        iv = jnp.where(c, iregs[j], iv)
        kregs[j] = nk
        iregs[j] = ni
    return kv, iv


def _sc_insert_tie(kregs, iregs, kv, iv):
    for j in range(K):
        c = (kv > kregs[j]) | ((kv == kregs[j]) & (iv < iregs[j]))
        nk = jnp.where(c, kv, kregs[j])
        kv = jnp.where(c, kregs[j], kv)
        ni = jnp.where(c, iv, iregs[j])
        iv = jnp.where(c, iregs[j], iv)
        kregs[j] = nk
        iregs[j] = ni
    return kv, iv


def _sc_topk_cands(sim_flat):
    from jax.experimental.pallas import tpu_sc as plsc

    mesh = plsc.VectorSubcoreMesh(core_axis_name="c", subcore_axis_name="s")

    @functools.partial(
        pl.kernel,
        out_type=[
            jax.ShapeDtypeStruct((NPAIR * K * 16,), jnp.float32),
            jax.ShapeDtypeStruct((NPAIR * K * 16,), jnp.int32),
        ],
        mesh=mesh,
        scratch_types=[
            pltpu.VMEM((ROWLEN,), jnp.float32),
            pltpu.VMEM((K * 16,), jnp.float32),
            pltpu.VMEM((K * 16,), jnp.int32),
        ],
    )
    def body(sim_hbm, outk_hbm, outi_hbm, row_v, kbuf_v, ibuf_v):
        wid = lax.axis_index("s") * 2 + lax.axis_index("c")

        @pl.when(wid < NPAIR)
        def _():
            pltpu.sync_copy(sim_hbm.at[pl.ds(wid * ROWLEN, ROWLEN)], row_v)
            lane = lax.iota(jnp.int32, 16)
            ninf = jnp.float32(-jnp.inf)

            NS = 4

            def step(g, carry):
                regs = list(carry)
                gk = [regs[2 * s * K:(2 * s + 1) * K] for s in range(NS)]
                gi = [regs[(2 * s + 1) * K:(2 * s + 2) * K] for s in range(NS)]
                base = g * 16 * NS
                for s in range(NS):
                    v = row_v[pl.ds(base + s * 16, 16)]
                    _sc_insert(gk[s], gi[s], v, base + s * 16 + lane)
                out = []
                for s in range(NS):
                    out += gk[s] + gi[s]
                return tuple(out)

            init = tuple(
                ([jnp.full((16,), ninf, jnp.float32) for _ in range(K)]
                 + [jnp.full((16,), _BIGI, jnp.int32) for _ in range(K)]) * NS)
            fin = list(lax.fori_loop(0, NVREG // NS, step, init))
            gk = [fin[2 * s * K:(2 * s + 1) * K] for s in range(NS)]
            gi = [fin[(2 * s + 1) * K:(2 * s + 2) * K] for s in range(NS)]
            g0k, g0i = gk[0], gi[0]
            for r in range((NVREG // NS) * NS, NVREG):
                tb = r * 16
                _sc_insert(g0k, g0i, row_v[pl.ds(tb, 16)], tb + lane)
            for s in range(1, NS):
                for j in range(K):
                    _sc_insert_tie(g0k, g0i, gk[s][j], gi[s][j])
            for j in range(K):
                kbuf_v[pl.ds(j * 16, 16)] = g0k[j]
                ibuf_v[pl.ds(j * 16, 16)] = g0i[j]
            pltpu.sync_copy(kbuf_v, outk_hbm.at[pl.ds(wid * K * 16, K * 16)])
            pltpu.sync_copy(ibuf_v, outi_hbm.at[pl.ds(wid * K * 16, K * 16)])

    return body(sim_flat)


def _tcb_body(y_ref, ck_ref, ci_ref, sp_ref, sc_ref, sh_ref, gw_ref, gb_ref,
              wu_ref, g2_ref, b2_ref, o_ref, zs_ref, st_ref):
    i = pl.program_id(0)

    @pl.when(i == 0)
    def _():
        st_ref[...] = jnp.zeros_like(st_ref)

    @pl.when(i < B)
    def _():
        _tcb_gcn(y_ref, ck_ref, ci_ref, sp_ref, sc_ref, sh_ref, gw_ref,
                 gb_ref, wu_ref, zs_ref, st_ref, i)

    @pl.when(i == B)
    def _():
        eps = jnp.float32(1e-5)
        mean2 = st_ref[:, 0:1] / CNT
        var2 = st_ref[:, 1:2] / CNT - mean2 * mean2
        s2v = lax.rsqrt(var2 + eps) * g2_ref[...]
        st_ref[:, 2:3] = s2v
        st_ref[:, 3:4] = b2_ref[...] - mean2 * s2v

    @pl.when(i >= B)
    def _():
        j = i - B
        o_ref[0] = (zs_ref[pl.ds(j, 1)][0] * st_ref[:, 2:3]
                    + st_ref[:, 3:4])


def _tcb_gcn(y_ref, ck_ref, ci_ref, sp_ref, sc_ref, sh_ref, gw_ref, gb_ref,
             wu_ref, zs_ref, st_ref, b):
    spanv = sp_ref[0, 0]
    ninf = -jnp.float32(jnp.inf)
    big = jnp.int32(_BIGI)
    k2 = ck_ref[0]
    i2 = ci_ref[0]
    k8iota = lax.broadcasted_iota(jnp.int32, (T1, K), 1)
    idxm = jnp.zeros((T1, K), jnp.int32)
    for k in range(K):
        m = jnp.max(k2, axis=1, keepdims=True)
        sel = k2 == m
        fidx = jnp.min(jnp.where(sel, i2, big), axis=1, keepdims=True)
        k2 = jnp.where(sel & (i2 == fidx), ninf, k2)
        idxm = jnp.where(k8iota == k, fidx, idxm)
    row = idxm // N
    col = idxm - row * N
    tcol = lax.broadcasted_iota(jnp.int32, (T1, K), 0)
    rowg = row + tcol * N
    colg = col + (tcol + spanv) * N
    lanes3 = lax.broadcasted_iota(jnp.int32, (T1, K, NODES), 2)
    s_rowg = jnp.where(lanes3 == rowg[:, :, None], 1.0, 0.0).reshape(
        T1 * K, NODES)
    s_colg = jnp.where(lanes3 == colg[:, :, None], 1.0, 0.0).reshape(
        T1 * K, NODES)
    s_src = jnp.concatenate([s_rowg, s_colg], axis=0)
    s_dst = jnp.concatenate([s_colg, s_rowg], axis=0)

    xn = y_ref[0] * sc_ref[...] + sh_ref[...]
    h = lax.dot_general(xn, gw_ref[...], (((1,), (1,)), ((), ())),
                        preferred_element_type=jnp.float32)

    ones_e = jnp.ones((E, 1), jnp.float32)
    deg = 1.0 + lax.dot_general(s_dst, ones_e, (((0,), (0,)), ((), ())),
                                preferred_element_type=jnp.float32)
    dinv = lax.rsqrt(deg)
    hs = lax.dot_general(s_src, h, (((1,), (0,)), ((), ())),
                         preferred_element_type=jnp.float32)
    dinv_src = lax.dot_general(s_src, dinv, (((1,), (0,)), ((), ())),
                               preferred_element_type=jnp.float32)
    dinv_dst = lax.dot_general(s_dst, dinv, (((1,), (0,)), ((), ())),
                               preferred_element_type=jnp.float32)
    contrib = hs * (dinv_src * dinv_dst)
    scat = lax.dot_general(s_dst, contrib, (((0,), (0,)), ((), ())),
                           preferred_element_type=jnp.float32)
    out = h * (dinv * dinv) + scat + gb_ref[...]

    wu = wu_ref[...]
    dn = (((1,), (1,)), ((), ()))
    for t in range(V):
        z = lax.dot_general(wu[1], out[t * N:(t + 1) * N, :], dn,
                            preferred_element_type=jnp.float32)
        if t > 0:
            z = z + lax.dot_general(wu[0], out[(t - 1) * N:t * N, :], dn,
                                    preferred_element_type=jnp.float32)
        if t < V - 1:
            z = z + lax.dot_general(wu[2], out[(t + 1) * N:(t + 2) * N, :], dn,
                                    preferred_element_type=jnp.float32)
        zs_ref[pl.ds(b * V + t, 1)] = z[None]
        st_ref[:, 0:1] += jnp.sum(z, axis=1, keepdims=True)
        st_ref[:, 1:2] += jnp.sum(z * z, axis=1, keepdims=True)


def _tcb(y_flat, ck, ci, span_arr, scale, shift, gcn_w, gcn_b, wu3, g2, b2):
    nsteps = B + B * V
    return pl.pallas_call(
        _tcb_body,
        grid=(nsteps,),
        in_specs=[
            pl.BlockSpec((1, NODES, C), lambda i: (jnp.minimum(i, B - 1), 0, 0)),
            pl.BlockSpec((1, T1, K * 16), lambda i: (jnp.minimum(i, B - 1), 0, 0)),
            pl.BlockSpec((1, T1, K * 16), lambda i: (jnp.minimum(i, B - 1), 0, 0)),
            pl.BlockSpec(memory_space=pltpu.SMEM),
            pl.BlockSpec((1, C), lambda i: (0, 0)),
            pl.BlockSpec((1, C), lambda i: (0, 0)),
            pl.BlockSpec((C, C), lambda i: (0, 0)),
            pl.BlockSpec((1, C), lambda i: (0, 0)),
            pl.BlockSpec((3, C, C), lambda i: (0, 0, 0)),
            pl.BlockSpec((C, 1), lambda i: (0, 0)),
            pl.BlockSpec((C, 1), lambda i: (0, 0)),
        ],
        out_specs=pl.BlockSpec((1, C, N), lambda i: (
            jnp.where(i < B, 0, i - B), 0, 0)),
        out_shape=jax.ShapeDtypeStruct((B * V, C, N), jnp.float32),
        scratch_shapes=[
            pltpu.VMEM((B * V, C, N), jnp.float32),
            pltpu.VMEM((C, 8), jnp.float32),
        ],
    )(y_flat, ck, ci, span_arr, scale, shift, gcn_w, gcn_b, wu3, g2, b2)


def kernel(x, w_down, bn1_g, bn1_b, gcn_w, gcn_b, w_up, bn2_g, bn2_b,
           batch: Any, span: Any):
    dep = jnp.asarray(batch, jnp.float32) / B
    x3 = x.reshape(B * V, C, N)
    w3 = jnp.transpose(w_down[:, :, :, 0, 0], (2, 0, 1)) * dep

    y, scale, shift, simn = _tca(x3, w3, bn1_g[None, :], bn1_b[None, :])
    ck, ci = _sc_topk_cands(simn.reshape(-1))
    ck = ck.reshape(B, T1, K * 16)
    ci = ci.reshape(B, T1, K * 16)
    span_arr = jnp.asarray(span, jnp.int32).reshape(1, 1)

    y_flat = y.reshape(B, NODES, C)
    wu3 = jnp.transpose(w_up[:, :, :, 0, 0], (2, 0, 1))
    out = _tcb(y_flat, ck, ci, span_arr, scale, shift, gcn_w, gcn_b[None, :],
               wu3, bn2_g[:, None], bn2_b[:, None])
    return out.reshape(B * V, C, 14, 14)


# speedup vs baseline: 1.1013x; 1.1013x over previous
import functools
from typing import Any

import jax
import jax.numpy as jnp
from jax import lax
from jax.experimental import pallas as pl
from jax.experimental.pallas import tpu as pltpu

B = 4
V = 8
C = 256
N = 196
T1 = V - 1
K = 8
NODES = V * N
E = 2 * T1 * K
CNT = B * V * N
NPAIR = B * T1
ROWLEN = N * N
NVREG = ROWLEN // 16
_BIGI = 2**30


def _tca_body(xm_ref, xc_ref, xp_ref, w_ref, g_ref, be_ref,
              y_ref, sco_ref, sho_ref, sm_ref, ys_ref, st_ref):
    i = pl.program_id(0)

    @pl.when(i == 0)
    def _():
        st_ref[...] = jnp.zeros_like(st_ref)

    @pl.when(i < B * V)
    def _():
        t = i % V
        xm = xm_ref[0]
        xc = xc_ref[0]
        xp = xp_ref[0]
        w = w_ref[...]
        dn = (((0,), (1,)), ((), ()))
        y = lax.dot_general(xc, w[1], dn, preferred_element_type=jnp.float32)
        ym = lax.dot_general(xm, w[0], dn, preferred_element_type=jnp.float32)
        yp = lax.dot_general(xp, w[2], dn, preferred_element_type=jnp.float32)
        mm = jnp.where(t > 0, 1.0, 0.0).astype(jnp.float32)
        mp = jnp.where(t < V - 1, 1.0, 0.0).astype(jnp.float32)
        y = y + mm * ym + mp * yp
        y_ref[0, 0] = y
        ys_ref[pl.ds(i, 1)] = y[None]
        st_ref[0:1, :] += jnp.sum(y, axis=0, keepdims=True)
        st_ref[1:2, :] += jnp.sum(y * y, axis=0, keepdims=True)

    @pl.when(i == B * V)
    def _():
        eps = jnp.float32(1e-5)
        mean = st_ref[0:1, :] / CNT
        var = st_ref[1:2, :] / CNT - mean * mean
        s = lax.rsqrt(var + eps) * g_ref[...]
        sh = be_ref[...] - mean * s
        st_ref[2:3, :] = s
        st_ref[3:4, :] = sh
        sco_ref[...] = s
        sho_ref[...] = sh

    @pl.when(i >= B * V)
    def _():
        p = i - B * V
        b = p // T1
        t = p % T1
        q = b * V + t
        s = st_ref[2:3, :]
        sh = st_ref[3:4, :]
        xa = ys_ref[pl.ds(q, 1)][0] * s + sh
        xb = ys_ref[pl.ds(q + 1, 1)][0] * s + sh
        a2 = jnp.sum(xa * xa, axis=1, keepdims=True)
        b2 = jnp.sum(xb * xb, axis=1, keepdims=True)
        g = lax.dot_general(xa, xb, (((1,), (1,)), ((), ())),
                            preferred_element_type=jnp.float32)
        d2 = jnp.clip(a2 + b2.T - 2.0 * g, 0.0, None)
        sim = -jnp.sqrt(d2)
        nrm = jnp.sqrt(jnp.sum(sim * sim))
        sm_ref[0] = sim / jnp.maximum(nrm, 1e-12)


def _tca(x3, w3, bn1_g2, bn1_b2):
    nsteps = B * V + NPAIR
    zero3 = lambda i: (0, 0, 0)
    return pl.pallas_call(
        _tca_body,
        grid=(nsteps,),
        in_specs=[
            pl.BlockSpec((1, C, N), lambda i: (
                jnp.where(i < B * V,
                          (i // V) * V + jnp.maximum(i % V - 1, 0), 0), 0, 0)),
            pl.BlockSpec((1, C, N), lambda i: (jnp.where(i < B * V, i, 0), 0, 0)),
            pl.BlockSpec((1, C, N), lambda i: (
                jnp.where(i < B * V,
                          (i // V) * V + jnp.minimum(i % V + 1, V - 1), 0), 0, 0)),
            pl.BlockSpec((3, C, C), zero3),
            pl.BlockSpec((1, C), lambda i: (0, 0)),
            pl.BlockSpec((1, C), lambda i: (0, 0)),
        ],
        out_specs=[
            pl.BlockSpec((1, 1, N, C), lambda i: (
                jnp.where(i < B * V, i // V, B - 1),
                jnp.where(i < B * V, i % V, V - 1), 0, 0)),
            pl.BlockSpec((1, C), lambda i: (0, 0)),
            pl.BlockSpec((1, C), lambda i: (0, 0)),
            pl.BlockSpec((1, N, N), lambda i: (
                jnp.where(i < B * V, 0, i - B * V), 0, 0)),
        ],
        out_shape=[
            jax.ShapeDtypeStruct((B, V, N, C), jnp.float32),
            jax.ShapeDtypeStruct((1, C), jnp.float32),
            jax.ShapeDtypeStruct((1, C), jnp.float32),
            jax.ShapeDtypeStruct((NPAIR, N, N), jnp.float32),
        ],
        scratch_shapes=[
            pltpu.VMEM((B * V, N, C), jnp.float32),
            pltpu.VMEM((8, C), jnp.float32),
        ],
    )(x3, x3, x3, w3, bn1_g2, bn1_b2)


def _sc_insert(kregs, iregs, kv, iv):
    for j in range(K):
        c = kv > kregs[j]
        nk = jnp.where(c, kv, kregs[j])
        kv = jnp.where(c, kregs[j], kv)
        ni = jnp.where(c, iv, iregs[j])
        iv = jnp.where(c, iregs[j], iv)
        kregs[j] = nk
        iregs[j] = ni
    return kv, iv


def _sc_insert_tie(kregs, iregs, kv, iv):
    for j in range(K):
        c = (kv > kregs[j]) | ((kv == kregs[j]) & (iv < iregs[j]))
        nk = jnp.where(c, kv, kregs[j])
        kv = jnp.where(c, kregs[j], kv)
        ni = jnp.where(c, iv, iregs[j])
        iv = jnp.where(c, iregs[j], iv)
        kregs[j] = nk
        iregs[j] = ni
    return kv, iv


def _sc_topk_cands(sim_flat):
    from jax.experimental.pallas import tpu_sc as plsc

    mesh = plsc.VectorSubcoreMesh(core_axis_name="c", subcore_axis_name="s")

    @functools.partial(
        pl.kernel,
        out_type=[
            jax.ShapeDtypeStruct((NPAIR * K * 16,), jnp.float32),
            jax.ShapeDtypeStruct((NPAIR * K * 16,), jnp.int32),
        ],
        mesh=mesh,
        scratch_types=[
            pltpu.VMEM((ROWLEN,), jnp.float32),
            pltpu.VMEM((K * 16,), jnp.float32),
            pltpu.VMEM((K * 16,), jnp.int32),
            pltpu.SemaphoreType.DMA,
            pltpu.SemaphoreType.DMA,
            pltpu.SemaphoreType.DMA,
            pltpu.SemaphoreType.DMA,
        ],
    )
    def body(sim_hbm, outk_hbm, outi_hbm, row_v, kbuf_v, ibuf_v,
             sem0, sem1, sem2, sem3):
        wid = lax.axis_index("s") * 2 + lax.axis_index("c")

        @pl.when(wid < NPAIR)
        def _():
            CH = 9600
            sems = [sem0, sem1, sem2, sem3]
            copies = []
            for c in range(4):
                ln = CH if c < 3 else (ROWLEN - 3 * CH)
                copies.append(pltpu.async_copy(
                    sim_hbm.at[pl.ds(wid * ROWLEN + c * CH, ln)],
                    row_v.at[pl.ds(c * CH, ln)], sems[c]))
            lane = lax.iota(jnp.int32, 16)
            ninf = jnp.float32(-jnp.inf)

            def step(g, carry):
                regs = list(carry)
                g0k, g0i = regs[0:K], regs[K:2 * K]
                g1k, g1i = regs[2 * K:3 * K], regs[3 * K:4 * K]
                base = g * 32
                va = row_v[pl.ds(base, 16)]
                vb = row_v[pl.ds(base + 16, 16)]
                _sc_insert(g0k, g0i, va, base + lane)
                _sc_insert(g1k, g1i, vb, base + 16 + lane)
                return tuple(g0k + g0i + g1k + g1i)

            carry = tuple(
                ([jnp.full((16,), ninf, jnp.float32) for _ in range(K)]
                 + [jnp.full((16,), _BIGI, jnp.int32) for _ in range(K)]) * 2)
            for c in range(4):
                copies[c].wait()
                carry = lax.fori_loop(c * 300, (c + 1) * 300, step, carry)
            fin = list(carry)
            g0k, g0i = fin[0:K], fin[K:2 * K]
            g1k, g1i = fin[2 * K:3 * K], fin[3 * K:4 * K]
            tb = (NVREG - 1) * 16
            _sc_insert(g0k, g0i, row_v[pl.ds(tb, 16)], tb + lane)
            for j in range(K):
                _sc_insert_tie(g0k, g0i, g1k[j], g1i[j])
            for j in range(K):
                kbuf_v[pl.ds(j * 16, 16)] = g0k[j]
                ibuf_v[pl.ds(j * 16, 16)] = g0i[j]
            pltpu.sync_copy(kbuf_v, outk_hbm.at[pl.ds(wid * K * 16, K * 16)])
            pltpu.sync_copy(ibuf_v, outi_hbm.at[pl.ds(wid * K * 16, K * 16)])

    return body(sim_flat)


def _tcb_body(y_ref, ck_ref, ci_ref, sp_ref, sc_ref, sh_ref, gw_ref, gb_ref,
              wu_ref, g2_ref, b2_ref, o_ref, zs_ref, st_ref):
    i = pl.program_id(0)

    @pl.when(i == 0)
    def _():
        st_ref[...] = jnp.zeros_like(st_ref)

    @pl.when(i < B)
    def _():
        _tcb_gcn(y_ref, ck_ref, ci_ref, sp_ref, sc_ref, sh_ref, gw_ref,
                 gb_ref, wu_ref, zs_ref, st_ref, i)

    @pl.when(i == B)
    def _():
        eps = jnp.float32(1e-5)
        mean2 = st_ref[:, 0:1] / CNT
        var2 = st_ref[:, 1:2] / CNT - mean2 * mean2
        s2v = lax.rsqrt(var2 + eps) * g2_ref[...]
        st_ref[:, 2:3] = s2v
        st_ref[:, 3:4] = b2_ref[...] - mean2 * s2v

    @pl.when(i >= B)
    def _():
        j = i - B
        o_ref[0] = (zs_ref[pl.ds(j, 1)][0] * st_ref[:, 2:3]
                    + st_ref[:, 3:4])


def _tcb_gcn(y_ref, ck_ref, ci_ref, sp_ref, sc_ref, sh_ref, gw_ref, gb_ref,
             wu_ref, zs_ref, st_ref, b):
    spanv = sp_ref[0, 0]
    ninf = -jnp.float32(jnp.inf)
    big = jnp.int32(_BIGI)
    k2 = ck_ref[0]
    i2 = ci_ref[0]
    k8iota = lax.broadcasted_iota(jnp.int32, (T1, K), 1)
    idxm = jnp.zeros((T1, K), jnp.int32)
    for k in range(K):
        m = jnp.max(k2, axis=1, keepdims=True)
        sel = k2 == m
        fidx = jnp.min(jnp.where(sel, i2, big), axis=1, keepdims=True)
        k2 = jnp.where(sel & (i2 == fidx), ninf, k2)
        idxm = jnp.where(k8iota == k, fidx, idxm)
    row = idxm // N
    col = idxm - row * N
    tcol = lax.broadcasted_iota(jnp.int32, (T1, K), 0)
    rowg = row + tcol * N
    colg = col + (tcol + spanv) * N
    lanes3 = lax.broadcasted_iota(jnp.int32, (T1, K, NODES), 2)
    s_rowg = jnp.where(lanes3 == rowg[:, :, None], 1.0, 0.0).reshape(
        T1 * K, NODES)
    s_colg = jnp.where(lanes3 == colg[:, :, None], 1.0, 0.0).reshape(
        T1 * K, NODES)
    s_src = jnp.concatenate([s_rowg, s_colg], axis=0)
    s_dst = jnp.concatenate([s_colg, s_rowg], axis=0)

    xn = y_ref[0] * sc_ref[...] + sh_ref[...]
    h = lax.dot_general(xn, gw_ref[...], (((1,), (1,)), ((), ())),
                        preferred_element_type=jnp.float32)

    ones_e = jnp.ones((E, 1), jnp.float32)
    deg = 1.0 + lax.dot_general(s_dst, ones_e, (((0,), (0,)), ((), ())),
                                preferred_element_type=jnp.float32)
    dinv = lax.rsqrt(deg)
    hs = lax.dot_general(s_src, h, (((1,), (0,)), ((), ())),
                         preferred_element_type=jnp.float32)
    dinv_src = lax.dot_general(s_src, dinv, (((1,), (0,)), ((), ())),
                               preferred_element_type=jnp.float32)
    dinv_dst = lax.dot_general(s_dst, dinv, (((1,), (0,)), ((), ())),
                               preferred_element_type=jnp.float32)
    contrib = hs * (dinv_src * dinv_dst)
    scat = lax.dot_general(s_dst, contrib, (((0,), (0,)), ((), ())),
                           preferred_element_type=jnp.float32)
    out = h * (dinv * dinv) + scat + gb_ref[...]

    wu = wu_ref[...]
    dn = (((1,), (1,)), ((), ()))
    for t in range(V):
        z = lax.dot_general(wu[1], out[t * N:(t + 1) * N, :], dn,
                            preferred_element_type=jnp.float32)
        if t > 0:
            z = z + lax.dot_general(wu[0], out[(t - 1) * N:t * N, :], dn,
                                    preferred_element_type=jnp.float32)
        if t < V - 1:
            z = z + lax.dot_general(wu[2], out[(t + 1) * N:(t + 2) * N, :], dn,
                                    preferred_element_type=jnp.float32)
        zs_ref[pl.ds(b * V + t, 1)] = z[None]
        st_ref[:, 0:1] += jnp.sum(z, axis=1, keepdims=True)
        st_ref[:, 1:2] += jnp.sum(z * z, axis=1, keepdims=True)


def _tcb(y_flat, ck, ci, span_arr, scale, shift, gcn_w, gcn_b, wu3, g2, b2):
    nsteps = B + B * V
    return pl.pallas_call(
        _tcb_body,
        grid=(nsteps,),
        in_specs=[
            pl.BlockSpec((1, NODES, C), lambda i: (jnp.minimum(i, B - 1), 0, 0)),
            pl.BlockSpec((1, T1, K * 16), lambda i: (jnp.minimum(i, B - 1), 0, 0)),
            pl.BlockSpec((1, T1, K * 16), lambda i: (jnp.minimum(i, B - 1), 0, 0)),
            pl.BlockSpec(memory_space=pltpu.SMEM),
            pl.BlockSpec((1, C), lambda i: (0, 0)),
            pl.BlockSpec((1, C), lambda i: (0, 0)),
            pl.BlockSpec((C, C), lambda i: (0, 0)),
            pl.BlockSpec((1, C), lambda i: (0, 0)),
            pl.BlockSpec((3, C, C), lambda i: (0, 0, 0)),
            pl.BlockSpec((C, 1), lambda i: (0, 0)),
            pl.BlockSpec((C, 1), lambda i: (0, 0)),
        ],
        out_specs=pl.BlockSpec((1, C, N), lambda i: (
            jnp.where(i < B, 0, i - B), 0, 0)),
        out_shape=jax.ShapeDtypeStruct((B * V, C, N), jnp.float32),
        scratch_shapes=[
            pltpu.VMEM((B * V, C, N), jnp.float32),
            pltpu.VMEM((C, 8), jnp.float32),
        ],
    )(y_flat, ck, ci, span_arr, scale, shift, gcn_w, gcn_b, wu3, g2, b2)


def kernel(x, w_down, bn1_g, bn1_b, gcn_w, gcn_b, w_up, bn2_g, bn2_b,
           batch: Any, span: Any):
    dep = jnp.asarray(batch, jnp.float32) / B
    x3 = x.reshape(B * V, C, N)
    w3 = jnp.transpose(w_down[:, :, :, 0, 0], (2, 0, 1)) * dep

    y, scale, shift, simn = _tca(x3, w3, bn1_g[None, :], bn1_b[None, :])
    ck, ci = _sc_topk_cands(simn.reshape(-1))
    ck = ck.reshape(B, T1, K * 16)
    ci = ci.reshape(B, T1, K * 16)
    span_arr = jnp.asarray(span, jnp.int32).reshape(1, 1)

    y_flat = y.reshape(B, NODES, C)
    wu3 = jnp.transpose(w_up[:, :, :, 0, 0], (2, 0, 1))
    out = _tcb(y_flat, ck, ci, span_arr, scale, shift, gcn_w, gcn_b[None, :],
               wu3, bn2_g[:, None], bn2_b[:, None])
    return out.reshape(B * V, C, 14, 14)
